# baseline (speedup 1.0000x reference)
import functools

import jax
import jax.numpy as jnp
from jax import lax
from jax.experimental import pallas as pl
from jax.experimental.pallas import tpu as pltpu
from jax.experimental.pallas import tpu_sc as plsc

N = 10000
E = 320000
D_IN = 128
D_H = 128
D_OUT = 64

NC = 2
NS = 16
NW = NC * NS
CH = 128

NCHA = -(-(E // NW) // CH)
EPTA = NCHA * CH

NCHD = -(-(E // NS) // CH)
EPTD = NCHD * CH

NP = N + 8
WR = 632
WR_LAST = N - (NS - 1) * WR
DEG_W = 16

_MESH = plsc.VectorSubcoreMesh(
    core_axis_name="c", subcore_axis_name="s", num_cores=NC, num_subcores=NS
)
_PARAMS = pltpu.CompilerParams(use_tc_tiling_on_sc=False)


@functools.partial(
    pl.kernel,
    out_type=(
        jax.ShapeDtypeStruct((N, DEG_W), jnp.float32),
        jax.ShapeDtypeStruct((N, DEG_W), jnp.float32),
    ),
    mesh=_MESH,
    compiler_params=_PARAMS,
    scratch_types=[
        pltpu.VMEM_SHARED((NP, DEG_W), jnp.float32),
        pltpu.VMEM((CH,), jnp.int32),
        pltpu.VMEM((CH,), jnp.int32),
        pltpu.VMEM((CH, DEG_W), jnp.float32),
        pltpu.SemaphoreType.DMA,
        pltpu.SemaphoreType.DMA,
    ],
)
def _deg_kernel(srcp, dstp, z16, ones, dout, din, acc, i0, i1, ones_v, semi0, semi1):
    c = lax.axis_index("c")
    s = lax.axis_index("s")
    ebase = s * EPTD
    pltpu.sync_copy(ones, ones_v)

    ibuf = (i0, i1)
    semi = (semi0, semi1)

    @pl.when(s == 0)
    def _():
        pltpu.sync_copy(z16, acc.at[pl.ds(0, N)])

    def run(arr, out):
        def load_idx(j, p):
            pltpu.async_copy(arr.at[pl.ds(ebase + j * CH, CH)], ibuf[p], semi[p])

        def wait_idx(j, p):
            pltpu.make_async_copy(
                arr.at[pl.ds(ebase + j * CH, CH)], ibuf[p], semi[p]
            ).wait()

        load_idx(0, 0)
        load_idx(1, 1)
        plsc.subcore_barrier()

        def body(j, _):
            for p in (0, 1):
                @pl.when(j % 2 == p)
                def _():
                    wait_idx(j, p)
                    pltpu.sync_copy(ones_v, acc.at[ibuf[p]], add=True)

                    @pl.when(j + 2 < NCHD)
                    def _():
                        load_idx(j + 2, p)

            return ()

        lax.fori_loop(0, NCHD, body, ())
        plsc.subcore_barrier()

        @pl.when(s < NS - 1)
        def _():
            sl = pl.ds(s * WR, WR)
            pltpu.sync_copy(acc.at[sl], out.at[sl])

        @pl.when(s == NS - 1)
        def _():
            sl = pl.ds(s * WR, WR_LAST)
            pltpu.sync_copy(acc.at[sl], out.at[sl])

    @pl.when(c == 0)
    def _():
        run(srcp, dout)

    @pl.when(c == 1)
    def _():
        run(dstp, din)


def _make_agg(D):
    @functools.partial(
        pl.kernel,
        out_type=jax.ShapeDtypeStruct((NC, N, D), jnp.float32),
        mesh=_MESH,
        compiler_params=_PARAMS,
        scratch_types=[
            pltpu.VMEM_SHARED((NP, D), jnp.float32),
            pltpu.VMEM((CH,), jnp.int32),
            pltpu.VMEM((CH,), jnp.int32),
            pltpu.VMEM((CH,), jnp.int32),
            pltpu.VMEM((CH,), jnp.int32),
            pltpu.VMEM((CH, D), jnp.float32),
            pltpu.VMEM((CH, D), jnp.float32),
            pltpu.SemaphoreType.DMA,
            pltpu.SemaphoreType.DMA,
            pltpu.SemaphoreType.DMA,
            pltpu.SemaphoreType.DMA,
        ],
    )
    def agg(h, srcp, dstp, zd, out, acc,
            is0, is1, id0, id1, rows0, rows1, semi0, semi1, semg0, semg1):
        c = lax.axis_index("c")
        s = lax.axis_index("s")
        wid = c * NS + s
        ebase = wid * EPTA

        isbuf = (is0, is1)
        idbuf = (id0, id1)
        rows = (rows0, rows1)
        semi = (semi0, semi1)
        semg = (semg0, semg1)

        def load_idx(j, p):
            pltpu.async_copy(srcp.at[pl.ds(ebase + j * CH, CH)], isbuf[p], semi[p])
            pltpu.async_copy(dstp.at[pl.ds(ebase + j * CH, CH)], idbuf[p], semi[p])

        def wait_idx(j, p):
            pltpu.make_async_copy(
                srcp.at[pl.ds(ebase + j * CH, CH)], isbuf[p], semi[p]
            ).wait()
            pltpu.make_async_copy(
                dstp.at[pl.ds(ebase + j * CH, CH)], idbuf[p], semi[p]
            ).wait()

        def gather(p):
            pltpu.async_copy(h.at[isbuf[p]], rows[p], semg[p])

        def wait_gather(p):
            pltpu.make_async_copy(h.at[isbuf[p]], rows[p], semg[p]).wait()

        @pl.when(s == 0)
        def _():
            pltpu.sync_copy(zd, acc.at[pl.ds(0, N)])

        load_idx(0, 0)
        load_idx(1, 1)
        plsc.subcore_barrier()
        wait_idx(0, 0)
        gather(0)

        def body(j, _):
            for p in (0, 1):
                @pl.when(j % 2 == p)
                def _():
                    q = 1 - p

                    @pl.when(j + 1 < NCHA)
                    def _():
                        wait_idx(j + 1, q)
                        gather(q)

                    wait_gather(p)
                    pltpu.sync_copy(rows[p], acc.at[idbuf[p]], add=True)

                    @pl.when(j + 2 < NCHA)
                    def _():
                        load_idx(j + 2, p)

            return ()

        lax.fori_loop(0, NCHA, body, ())
        plsc.subcore_barrier()

        @pl.when(s < NS - 1)
        def _():
            sl = pl.ds(s * WR, WR)
            pltpu.sync_copy(acc.at[sl], out.at[c, sl])

        @pl.when(s == NS - 1)
        def _():
            sl = pl.ds(s * WR, WR_LAST)
            pltpu.sync_copy(acc.at[sl], out.at[c, sl])

    return agg


_agg128 = _make_agg(D_H)
_agg64 = _make_agg(D_OUT)


_BLK = 1000


def _norm_from(dp):
    return lax.rsqrt(jnp.maximum(dp[:, 0], 1.0))


def _matmul(x, w):
    m, k = x.shape
    d = w.shape[1]

    def body(x_ref, w_ref, o_ref):
        o_ref[...] = jnp.dot(x_ref[...], w_ref[...], preferred_element_type=jnp.float32)

    return pl.pallas_call(
        body,
        grid=(m // _BLK,),
        in_specs=[
            pl.BlockSpec((_BLK, k), lambda i: (i, 0)),
            pl.BlockSpec((k, d), lambda i: (0, 0)),
        ],
        out_specs=pl.BlockSpec((_BLK, d), lambda i: (i, 0)),
        out_shape=jax.ShapeDtypeStruct((m, d), jnp.float32),
    )(x, w)


def _scale(u, deg_out):
    m, d = u.shape

    def body(u_ref, dp_ref, o_ref):
        o_ref[...] = u_ref[...] * _norm_from(dp_ref[...])[:, None]

    return pl.pallas_call(
        body,
        grid=(m // _BLK,),
        in_specs=[
            pl.BlockSpec((_BLK, d), lambda i: (i, 0)),
            pl.BlockSpec((_BLK, DEG_W), lambda i: (i, 0)),
        ],
        out_specs=pl.BlockSpec((_BLK, d), lambda i: (i, 0)),
        out_shape=jax.ShapeDtypeStruct((m, d), jnp.float32),
    )(u, deg_out)


def _layer2_in(aggp, deg_in, deg_out, b1r, w2):
    d = w2.shape[1]

    def body(ap_ref, di_ref, do_ref, b_ref, w_ref, o_ref):
        ap = ap_ref[...]
        agg = ap[0] + ap[1]
        z = agg * _norm_from(di_ref[...])[:, None] + b_ref[...]
        z = jnp.maximum(z, 0.0)
        z = z * _norm_from(do_ref[...])[:, None]
        o_ref[...] = jnp.dot(z, w_ref[...], preferred_element_type=jnp.float32)

    return pl.pallas_call(
        body,
        grid=(N // _BLK,),
        in_specs=[
            pl.BlockSpec((NC, _BLK, D_H), lambda i: (0, i, 0)),
            pl.BlockSpec((_BLK, DEG_W), lambda i: (i, 0)),
            pl.BlockSpec((_BLK, DEG_W), lambda i: (i, 0)),
            pl.BlockSpec((1, D_H), lambda i: (0, 0)),
            pl.BlockSpec((D_H, d), lambda i: (0, 0)),
        ],
        out_specs=pl.BlockSpec((_BLK, d), lambda i: (i, 0)),
        out_shape=jax.ShapeDtypeStruct((N, d), jnp.float32),
    )(aggp, deg_in, deg_out, b1r, w2)


def _final(aggp, deg_in, b2r):
    def body(ap_ref, di_ref, b_ref, o_ref):
        ap = ap_ref[...]
        agg = ap[0] + ap[1]
        o_ref[...] = agg * _norm_from(di_ref[...])[:, None] + b_ref[...]

    return pl.pallas_call(
        body,
        grid=(N // _BLK,),
        in_specs=[
            pl.BlockSpec((NC, _BLK, D_OUT), lambda i: (0, i, 0)),
            pl.BlockSpec((_BLK, DEG_W), lambda i: (i, 0)),
            pl.BlockSpec((1, D_OUT), lambda i: (0, 0)),
        ],
        out_specs=pl.BlockSpec((_BLK, D_OUT), lambda i: (i, 0)),
        out_shape=jax.ShapeDtypeStruct((N, D_OUT), jnp.float32),
    )(aggp, deg_in, b2r)


def _pad_tiles(arr, n_tiles, padded_len, fill):
    a2 = arr.reshape(n_tiles, E // n_tiles)
    a2 = jnp.pad(a2, ((0, 0), (0, padded_len - E // n_tiles)), constant_values=fill)
    return a2.reshape(-1)


def kernel(in_feat, edge_index, W1, b1, W2, b2):
    src, dst = edge_index[0], edge_index[1]
    src_a = _pad_tiles(src, NW, EPTA, 0)
    dst_a = _pad_tiles(dst, NW, EPTA, N)
    src_d = _pad_tiles(src, NS, EPTD, N)
    dst_d = _pad_tiles(dst, NS, EPTD, N)

    z16 = jnp.zeros((N, DEG_W), jnp.float32)
    ones = jnp.ones((CH, DEG_W), jnp.float32)
    z128 = jnp.zeros((N, D_H), jnp.float32)
    z64 = jnp.zeros((N, D_OUT), jnp.float32)

    deg_out, deg_in = _deg_kernel(src_d, dst_d, z16, ones)
    u = _matmul(in_feat, W1)
    h1 = _scale(u, deg_out)
    agg1 = _agg128(h1, src_a, dst_a, z128)
    h2 = _layer2_in(agg1, deg_in, deg_out, b1.reshape(1, D_H), W2)
    agg2 = _agg64(h2, src_a, dst_a, z64)
    return _final(agg2, deg_in, b2.reshape(1, D_OUT))
